# baseline (speedup 1.0000x reference)
import jax
import jax.numpy as jnp
from jax.experimental import pallas as pl
from jax.experimental.pallas import tpu as pltpu

ROWS_PER_BLOCK = 25000
SUB = 1000


def _fused_mlp_block(x_ref, w1t_ref, b1_ref, gamma_ref, beta_ref, w2t_ref,
                     b2_ref, out_ref):
    w1t = w1t_ref[...]
    w2t = w2t_ref[...]
    b1 = b1_ref[...]
    gamma = gamma_ref[...]
    beta = beta_ref[...]
    b2 = b2_ref[...]

    def body(j, carry):
        r = j * SUB
        h = jnp.dot(x_ref[pl.ds(r, SUB), :], w1t,
                    preferred_element_type=jnp.float32)
        h = h + b1
        h = jnp.where(h >= 0, h, 0.01 * h)
        mu = jnp.mean(h, axis=-1, keepdims=True)
        var = jnp.mean((h - mu) ** 2, axis=-1, keepdims=True)
        h = (h - mu) * jax.lax.rsqrt(var + 1e-5) * gamma + beta
        out = jnp.dot(h, w2t, preferred_element_type=jnp.float32)
        out_ref[pl.ds(r, SUB), :] = out + b2
        return carry

    jax.lax.fori_loop(0, ROWS_PER_BLOCK // SUB, body, 0)


@jax.jit
def kernel(x, W1, b1, gamma, beta, W2, b2):
    n, din = x.shape
    hid = W1.shape[0]
    dout = W2.shape[0]
    blk = ROWS_PER_BLOCK
    grid = (n // blk,)

    w1t = W1.T
    w2t = W2.T
    b1r = b1.reshape(1, hid)
    gammar = gamma.reshape(1, hid)
    betar = beta.reshape(1, hid)
    b2r = b2.reshape(1, dout)

    rep = lambda shape: pl.BlockSpec(shape, lambda i: (0, 0))
    return pl.pallas_call(
        _fused_mlp_block,
        grid=grid,
        in_specs=[
            pl.BlockSpec((blk, din), lambda i: (i, 0)),
            rep((din, hid)),
            rep((1, hid)),
            rep((1, hid)),
            rep((1, hid)),
            rep((hid, dout)),
            rep((1, dout)),
        ],
        out_specs=pl.BlockSpec((blk, dout), lambda i: (i, 0)),
        out_shape=jax.ShapeDtypeStruct((n, dout), jnp.float32),
        compiler_params=pltpu.CompilerParams(
            dimension_semantics=("arbitrary",),
            vmem_limit_bytes=127 * 1024 * 1024,
        ),
    )(x, w1t, b1r, gammar, betar, w2t, b2r)


# speedup vs baseline: 1.8510x; 1.8510x over previous
import jax
import jax.numpy as jnp
from jax.experimental import pallas as pl
from jax.experimental.pallas import tpu as pltpu

ROWS_PER_BLOCK = 25000


def _fused_mlp_block(x_ref, w1t_ref, b1_ref, gamma_ref, beta_ref, w2t_ref,
                     b2_ref, out_ref):
    h = jnp.dot(x_ref[...], w1t_ref[...], preferred_element_type=jnp.float32)
    h = h + b1_ref[...]
    h = jnp.where(h >= 0, h, 0.01 * h)
    mu = jnp.mean(h, axis=-1, keepdims=True)
    var = jnp.mean((h - mu) ** 2, axis=-1, keepdims=True)
    h = (h - mu) * jax.lax.rsqrt(var + 1e-5) * gamma_ref[...] + beta_ref[...]
    out = jnp.dot(h, w2t_ref[...], preferred_element_type=jnp.float32)
    out_ref[...] = out + b2_ref[...]


@jax.jit
def kernel(x, W1, b1, gamma, beta, W2, b2):
    n, din = x.shape
    hid = W1.shape[0]
    dout = W2.shape[0]
    blk = ROWS_PER_BLOCK
    grid = (n // blk,)

    w1t = W1.T
    w2t = W2.T
    b1r = b1.reshape(1, hid)
    gammar = gamma.reshape(1, hid)
    betar = beta.reshape(1, hid)
    b2r = b2.reshape(1, dout)

    rep = lambda shape: pl.BlockSpec(shape, lambda i: (0, 0))
    return pl.pallas_call(
        _fused_mlp_block,
        grid=grid,
        in_specs=[
            pl.BlockSpec((blk, din), lambda i: (i, 0)),
            rep((din, hid)),
            rep((1, hid)),
            rep((1, hid)),
            rep((1, hid)),
            rep((hid, dout)),
            rep((1, dout)),
        ],
        out_specs=pl.BlockSpec((blk, dout), lambda i: (i, 0)),
        out_shape=jax.ShapeDtypeStruct((n, dout), jnp.float32),
        compiler_params=pltpu.CompilerParams(
            dimension_semantics=("arbitrary",),
            vmem_limit_bytes=127 * 1024 * 1024,
        ),
    )(x, w1t, b1r, gammar, betar, w2t, b2r)


# speedup vs baseline: 1.9953x; 1.0780x over previous
import jax
import jax.numpy as jnp
from jax.experimental import pallas as pl
from jax.experimental.pallas import tpu as pltpu

ROWS_PER_BLOCK = 25000


def _fused_mlp_block(x_ref, w1t_ref, b1_ref, gamma_ref, beta_ref, w2t_ref,
                     b2_ref, out_ref):
    h = jax.lax.dot_general(x_ref[...], w1t_ref[...],
                            (((1,), (1,)), ((), ())),
                            preferred_element_type=jnp.float32)
    h = h + b1_ref[...]
    h = jnp.where(h >= 0, h, 0.01 * h)
    mu = jnp.mean(h, axis=-1, keepdims=True)
    var = jnp.mean((h - mu) ** 2, axis=-1, keepdims=True)
    h = (h - mu) * jax.lax.rsqrt(var + 1e-5) * gamma_ref[...] + beta_ref[...]
    out = jax.lax.dot_general(h, w2t_ref[...],
                              (((1,), (1,)), ((), ())),
                              preferred_element_type=jnp.float32)
    out_ref[...] = out + b2_ref[...]


@jax.jit
def kernel(x, W1, b1, gamma, beta, W2, b2):
    n, din = x.shape
    hid = W1.shape[0]
    dout = W2.shape[0]
    blk = ROWS_PER_BLOCK
    grid = (n // blk,)

    b1r = b1.reshape(1, hid)
    gammar = gamma.reshape(1, hid)
    betar = beta.reshape(1, hid)
    b2r = b2.reshape(1, dout)

    rep = lambda shape: pl.BlockSpec(shape, lambda i: (0, 0))
    return pl.pallas_call(
        _fused_mlp_block,
        grid=grid,
        in_specs=[
            pl.BlockSpec((blk, din), lambda i: (i, 0)),
            rep((din, hid)),
            rep((1, hid)),
            rep((1, hid)),
            rep((1, hid)),
            rep((hid, dout)),
            rep((1, dout)),
        ],
        out_specs=pl.BlockSpec((blk, dout), lambda i: (i, 0)),
        out_shape=jax.ShapeDtypeStruct((n, dout), jnp.float32),
        compiler_params=pltpu.CompilerParams(
            dimension_semantics=("arbitrary",),
            vmem_limit_bytes=127 * 1024 * 1024,
        ),
    )(x, W1, b1r, gammar, betar, W2, b2r)
